# baseline (speedup 1.0000x reference)
import functools

import jax
import jax.numpy as jnp
from jax.experimental import pallas as pl
from jax.experimental.pallas import tpu as pltpu

T_DIM = 4096
S_DIM = 2048
B_DIM = 1024
N_TOK = 16
N_EXP = 8
TOP_K = 2
G_DIM = 512
N_HEADS = 8
HD = S_DIM // N_HEADS
SEQ = 2048
BATCH = 2

BS = 128
NS = SEQ // BS
N_PAIR = 2 * (1 + TOP_K)
GEN_ID = N_EXP

_DN = (((1,), (1,)), ((), ()))

f32 = jnp.float32
bf16 = jnp.bfloat16


def _gelu_exact(x):
    return 0.5 * x * (1.0 + jax.lax.erf(x * 0.7071067811865476))


def _ln_f32(x, g, b, eps=1e-5):
    mu = jnp.mean(x, axis=-1, keepdims=True)
    var = jnp.mean((x - mu) ** 2, axis=-1, keepdims=True)
    return (x - mu) / jnp.sqrt(var + eps) * g + b


RBS = 256
NSR = SEQ // RBS


def _router_kernel(h_ref, rw1_ref, rb1_ref, rw2_ref, rb2_ref,
                   probs_ref, w_ref, idx_ref, psum_scr):
    s = pl.program_id(1)

    @pl.when(s == 0)
    def _init():
        psum_scr[...] = jnp.zeros((1, T_DIM), f32)

    psum_scr[...] += jnp.sum(h_ref[0], axis=0, keepdims=True)

    @pl.when(s == NSR - 1)
    def _finish():
        _router_tail(psum_scr[...] / SEQ, rw1_ref, rb1_ref, rw2_ref, rb2_ref,
                     probs_ref, w_ref, idx_ref)


def _router_tail(pooled, rw1_ref, rb1_ref, rw2_ref, rb2_ref,
                 probs_ref, w_ref, idx_ref):
    hid = jax.lax.dot_general(pooled, rw1_ref[...], _DN,
                              preferred_element_type=f32) + rb1_ref[...]
    hid = _gelu_exact(hid)
    logits = jax.lax.dot_general(hid, rw2_ref[...], _DN,
                                 preferred_element_type=f32) + rb2_ref[...]
    z = logits - jnp.max(logits, axis=-1, keepdims=True)
    ez = jnp.exp(z)
    probs = ez / jnp.sum(ez, axis=-1, keepdims=True)
    probs_ref[0] = probs

    ids = jax.lax.broadcasted_iota(jnp.int32, (1, N_EXP), 1)
    m1 = jnp.max(probs)
    i1 = jnp.min(jnp.where(probs == m1, ids, N_EXP))
    probs2 = jnp.where(ids == i1, -jnp.inf, probs)
    m2 = jnp.max(probs2)
    i2 = jnp.min(jnp.where(probs2 == m2, ids, N_EXP))
    denom = m1 + m2 + 1e-8
    pick = jax.lax.broadcasted_iota(jnp.int32, (1, TOP_K), 1)
    w_ref[0] = jnp.where(pick == 0, m1, m2) / denom
    idx_ref[0] = jnp.where(pick == 0, i1, i2).astype(jnp.int32)


def _run_router(h, rW1, rb1, rW2, rb2):
    probs, w, idx = pl.pallas_call(
        _router_kernel,
        grid=(BATCH, NSR),
        in_specs=[
            pl.BlockSpec((1, RBS, T_DIM), lambda b, s: (b, s, 0)),
            pl.BlockSpec((G_DIM, T_DIM), lambda b, s: (0, 0)),
            pl.BlockSpec((1, G_DIM), lambda b, s: (0, 0)),
            pl.BlockSpec((N_EXP, G_DIM), lambda b, s: (0, 0)),
            pl.BlockSpec((1, N_EXP), lambda b, s: (0, 0)),
        ],
        out_specs=[
            pl.BlockSpec((1, 1, N_EXP), lambda b, s: (b, 0, 0)),
            pl.BlockSpec((1, 1, TOP_K), lambda b, s: (b, 0, 0)),
            pl.BlockSpec((1, 1, TOP_K), lambda b, s: (b, 0, 0)),
        ],
        out_shape=[
            jax.ShapeDtypeStruct((BATCH, 1, N_EXP), f32),
            jax.ShapeDtypeStruct((BATCH, 1, TOP_K), f32),
            jax.ShapeDtypeStruct((BATCH, 1, TOP_K), jnp.int32),
        ],
        scratch_shapes=[pltpu.VMEM((1, T_DIM), f32)],
        compiler_params=pltpu.CompilerParams(
            dimension_semantics=("arbitrary", "arbitrary")),
    )(h, rW1, rb1.reshape(1, G_DIM), rW2, rb2.reshape(1, N_EXP))
    return probs.reshape(BATCH, N_EXP), w.reshape(BATCH, TOP_K), \
        idx.reshape(BATCH, TOP_K)


def _p_kernel(q_ref, bq_ref, wq_ref, wk_ref, p_ref):
    qq = jax.lax.dot_general(q_ref[...].astype(bf16),
                             wq_ref[...].astype(bf16), _DN,
                             preferred_element_type=f32) + bq_ref[...]
    rows = []
    for h_i in range(N_HEADS):
        sl = slice(h_i * HD, (h_i + 1) * HD)
        rows.append(jnp.dot(qq[:, sl].astype(bf16),
                            wk_ref[sl, :].astype(bf16),
                            preferred_element_type=f32))
    p_ref[...] = jnp.concatenate(rows, axis=0).astype(bf16)


def _run_p(q, bq, Win):
    return pl.pallas_call(
        _p_kernel,
        grid=(1,),
        in_specs=[
            pl.BlockSpec((N_TOK, S_DIM), lambda i: (0, 0)),
            pl.BlockSpec((1, S_DIM), lambda i: (0, 0)),
            pl.BlockSpec((S_DIM, S_DIM), lambda i: (0, 0)),
            pl.BlockSpec((S_DIM, S_DIM), lambda i: (1, 0)),
        ],
        out_specs=pl.BlockSpec((N_HEADS * N_TOK, S_DIM), lambda i: (0, 0)),
        out_shape=jax.ShapeDtypeStruct((N_HEADS * N_TOK, S_DIM), bf16),
        compiler_params=pltpu.CompilerParams(
            dimension_semantics=("arbitrary",)),
    )(q, bq, Win, Win)


def _mlp_kernel(se_ref, sb_ref, h_ref, *refs):
    wd_hbm = refs[0:9]
    wu_ref, bd_ref, bu_ref, lng_ref, lnb_ref, x2_ref = refs[9:15]
    land_wd, wdb, sem_wd = refs[15:18]
    p = pl.program_id(0)
    s = pl.program_id(1)

    def start(pair):
        for e in range(N_EXP + 1):
            @pl.when(se_ref[pair] == e)
            def _():
                pltpu.make_async_copy(wd_hbm[e], land_wd, sem_wd).start()

    def wait(pair):
        for e in range(N_EXP + 1):
            @pl.when(se_ref[pair] == e)
            def _():
                pltpu.make_async_copy(wd_hbm[e], land_wd, sem_wd).wait()

    @pl.when(s == 0)
    def _load_weights():
        @pl.when(p == 0)
        def _first():
            start(0)

        wait(p)
        wdb[...] = land_wd[...].astype(bf16)

        @pl.when(p < N_PAIR - 1)
        def _prefetch_next():
            start(p + 1)

    hb = h_ref[0].astype(bf16)
    x1 = jax.lax.dot_general(hb, wdb[...], _DN,
                             preferred_element_type=f32) + bd_ref[0]
    x1 = _gelu_exact(x1)
    x2 = jax.lax.dot_general(x1.astype(bf16), wu_ref[0], _DN,
                             preferred_element_type=f32) + bu_ref[0]
    x2 = _ln_f32(x2, lng_ref[0], lnb_ref[0])
    x2_ref[0] = x2.astype(bf16)


def _run_mlp(pair_expert, pair_batch, h, Wd_list, Wu_all,
             bd_all, bu_all, lng_all, lnb_all):
    any_spec = pl.BlockSpec(memory_space=pl.ANY)
    grid_spec = pltpu.PrefetchScalarGridSpec(
        num_scalar_prefetch=2,
        grid=(N_PAIR, NS),
        in_specs=(
            [pl.BlockSpec((1, BS, T_DIM), lambda p, s, se, sb: (sb[p], s, 0))]
            + [any_spec] * 9
            + [pl.BlockSpec((1, S_DIM, B_DIM),
                            lambda p, s, se, sb: (se[p], 0, 0)),
               pl.BlockSpec((1, 1, B_DIM), lambda p, s, se, sb: (se[p], 0, 0)),
               pl.BlockSpec((1, 1, S_DIM), lambda p, s, se, sb: (se[p], 0, 0)),
               pl.BlockSpec((1, 1, S_DIM), lambda p, s, se, sb: (se[p], 0, 0)),
               pl.BlockSpec((1, 1, S_DIM), lambda p, s, se, sb: (se[p], 0, 0))]
        ),
        out_specs=pl.BlockSpec((1, BS, S_DIM),
                               lambda p, s, se, sb: (p, s, 0)),
        scratch_shapes=[
            pltpu.VMEM((B_DIM, T_DIM), f32),
            pltpu.VMEM((B_DIM, T_DIM), bf16),
            pltpu.SemaphoreType.DMA,
        ],
    )
    return pl.pallas_call(
        _mlp_kernel,
        grid_spec=grid_spec,
        out_shape=jax.ShapeDtypeStruct((N_PAIR, SEQ, S_DIM), bf16),
        compiler_params=pltpu.CompilerParams(
            dimension_semantics=("arbitrary", "arbitrary")),
    )(pair_expert, pair_batch, h, *Wd_list,
      Wu_all, bd_all, bu_all, lng_all, lnb_all)


def _attn_kernel(se_ref, sb_ref, w_ref,
                 x2_ref, p_ref, wv_ref, bv_ref, wo_ref, bo_ref, q_ref,
                 png_ref, pnb_ref, out_ref):
    p = pl.program_id(0)
    x2c = x2_ref[0]
    sc = jax.lax.dot_general(p_ref[0], x2c, _DN,
                             preferred_element_type=f32)
    ctx_heads = []
    for h_i in range(N_HEADS):
        att = jax.nn.softmax(sc[h_i * N_TOK:(h_i + 1) * N_TOK, :] / 16.0,
                             axis=-1)
        t = jax.lax.dot_general(att.astype(bf16), x2c,
                                (((1,), (0,)), ((), ())),
                                preferred_element_type=f32)
        wv_h = wv_ref[0][h_i * HD:(h_i + 1) * HD, :]
        ctx_heads.append(
            jax.lax.dot_general(t.astype(bf16), wv_h, _DN,
                                preferred_element_type=f32)
            + bv_ref[0][:, h_i * HD:(h_i + 1) * HD])
    ctx = jnp.concatenate(ctx_heads, axis=-1)
    val = jax.lax.dot_general(ctx.astype(bf16), wo_ref[0], _DN,
                              preferred_element_type=f32) + bo_ref[0]
    val = _ln_f32(val + q_ref[0], png_ref[0], pnb_ref[0]) * w_ref[p]

    first = (p != 3) & (p != 5)

    @pl.when(first)
    def _set():
        out_ref[0, 0] = val

    @pl.when(jnp.logical_not(first))
    def _acc():
        out_ref[0, 0] += val


def _run_attn(pair_expert, pair_batch, pair_w, x2_all, P_all,
              Wv_all, bv_all, Wo_all, bo_all, q_all, png_all, pnb_all):
    grid_spec = pltpu.PrefetchScalarGridSpec(
        num_scalar_prefetch=3,
        grid=(N_PAIR,),
        in_specs=[
            pl.BlockSpec((1, SEQ, S_DIM), lambda p, se, sb, w: (p, 0, 0)),
            pl.BlockSpec((1, N_HEADS * N_TOK, S_DIM),
                         lambda p, se, sb, w: (se[p], 0, 0)),
            pl.BlockSpec((1, S_DIM, S_DIM),
                         lambda p, se, sb, w: (se[p], 0, 0)),
            pl.BlockSpec((1, 1, S_DIM), lambda p, se, sb, w: (se[p], 0, 0)),
            pl.BlockSpec((1, S_DIM, S_DIM),
                         lambda p, se, sb, w: (se[p], 0, 0)),
            pl.BlockSpec((1, 1, S_DIM), lambda p, se, sb, w: (se[p], 0, 0)),
            pl.BlockSpec((1, N_TOK, S_DIM),
                         lambda p, se, sb, w: (se[p], 0, 0)),
            pl.BlockSpec((1, 1, S_DIM), lambda p, se, sb, w: (se[p], 0, 0)),
            pl.BlockSpec((1, 1, S_DIM), lambda p, se, sb, w: (se[p], 0, 0)),
        ],
        out_specs=pl.BlockSpec(
            (1, 1, N_TOK, S_DIM),
            lambda p, se, sb, w: (sb[p], jnp.where(se[p] == GEN_ID, 0, 1),
                                  0, 0)),
    )
    out4 = pl.pallas_call(
        _attn_kernel,
        grid_spec=grid_spec,
        out_shape=jax.ShapeDtypeStruct((BATCH, 2, N_TOK, S_DIM), f32),
        compiler_params=pltpu.CompilerParams(
            dimension_semantics=("arbitrary",)),
    )(pair_expert, pair_batch, pair_w, x2_all, P_all,
      Wv_all, bv_all, Wo_all, bo_all, q_all, png_all, pnb_all)
    return out4.reshape(BATCH, 2 * N_TOK, S_DIM)


def gated_w_stack(name, idx, allp):
    return jnp.stack([p[name] for p in allp]).astype(bf16)

def kernel(h_teacher, params):
    gen = params['gen']
    allp = list(params['spec']) + [gen]

    def stack(name, dtype=None, bias=False):
        a = jnp.stack([p[name] for p in allp])
        if bias:
            a = a[:, None, :]
        return a.astype(dtype) if dtype is not None else a

    Wd_list = [p['Wd'] for p in allp]
    Wu_all = jnp.stack([p['Wu'] for p in allp]).astype(bf16)
    bd_all = stack('bd', bias=True)
    bu_all = stack('bu', bias=True)
    lng_all = stack('ln_g', bias=True)
    lnb_all = stack('ln_b', bias=True)
    Wv_all = jnp.stack(
        [p['Win'][2 * S_DIM:3 * S_DIM] for p in allp]).astype(bf16)
    bv_all = jnp.stack([p['bin'][2 * S_DIM:3 * S_DIM] for p in allp])[:, None]
    q_all = jnp.stack([p['q'][0] for p in allp])
    Wo_all = stack('Wo', bf16)
    bo_all = stack('bo', bias=True)
    png_all = stack('pn_g', bias=True)
    pnb_all = stack('pn_b', bias=True)

    probs, w, idx = _run_router(h_teacher, params['rW1'], params['rb1'],
                                params['rW2'], params['rb2'])

    gen_i = jnp.full((1,), GEN_ID, jnp.int32)
    one_w = jnp.ones((1,), f32)
    pair_expert = jnp.concatenate([gen_i, gen_i, idx[0], idx[1]])
    pair_batch = jnp.array([0, 1, 0, 0, 1, 1], jnp.int32)
    pair_w = jnp.concatenate([one_w, one_w, w[0], w[1]])

    def p_for(p_):
        return _run_p(p_['q'][0], p_['bin'][None, :S_DIM], p_['Win'])

    P_list = []
    for e, p_ in enumerate(allp):
        if e == GEN_ID:
            P_list.append(p_for(p_))
        else:
            used = jnp.any(idx == e)
            P_list.append(jax.lax.cond(
                used,
                lambda p_=p_: p_for(p_),
                lambda: jnp.zeros((N_HEADS * N_TOK, S_DIM), bf16)))
    P_all = jnp.stack(P_list)

    x2_all = _run_mlp(pair_expert, pair_batch, h_teacher,
                      Wd_list, Wu_all, bd_all, bu_all, lng_all, lnb_all)
    c_agg = _run_attn(pair_expert, pair_batch, pair_w, x2_all, P_all,
                      Wv_all, bv_all, Wo_all, bo_all, q_all, png_all, pnb_all)
    return c_agg, probs


# speedup vs baseline: 1.3203x; 1.3203x over previous
import functools

import jax
import jax.numpy as jnp
from jax.experimental import pallas as pl
from jax.experimental.pallas import tpu as pltpu

T_DIM = 4096
S_DIM = 2048
B_DIM = 1024
N_TOK = 16
N_EXP = 8
TOP_K = 2
G_DIM = 512
N_HEADS = 8
HD = S_DIM // N_HEADS
SEQ = 2048
BATCH = 2

BS = 512
NS = SEQ // BS
N_PAIR = 2 * (1 + TOP_K)
GEN_ID = N_EXP

_DN = (((1,), (1,)), ((), ()))

f32 = jnp.float32
bf16 = jnp.bfloat16


def _gelu_exact(x):
    return 0.5 * x * (1.0 + jax.lax.erf(x * 0.7071067811865476))


def _ln_f32(x, g, b, eps=1e-5):
    mu = jnp.mean(x, axis=-1, keepdims=True)
    var = jnp.mean((x - mu) ** 2, axis=-1, keepdims=True)
    return (x - mu) / jnp.sqrt(var + eps) * g + b


RBS = 256
NSR = SEQ // RBS


def _router_kernel(h_ref, rw1_ref, rb1_ref, rw2_ref, rb2_ref,
                   probs_ref, w_ref, idx_ref, psum_scr):
    s = pl.program_id(1)

    @pl.when(s == 0)
    def _init():
        psum_scr[...] = jnp.zeros((1, T_DIM), f32)

    psum_scr[...] += jnp.sum(h_ref[0], axis=0, keepdims=True)

    @pl.when(s == NSR - 1)
    def _finish():
        _router_tail(psum_scr[...] / SEQ, rw1_ref, rb1_ref, rw2_ref, rb2_ref,
                     probs_ref, w_ref, idx_ref)


def _router_tail(pooled, rw1_ref, rb1_ref, rw2_ref, rb2_ref,
                 probs_ref, w_ref, idx_ref):
    hid = jax.lax.dot_general(pooled, rw1_ref[...], _DN,
                              preferred_element_type=f32) + rb1_ref[...]
    hid = _gelu_exact(hid)
    logits = jax.lax.dot_general(hid, rw2_ref[...], _DN,
                                 preferred_element_type=f32) + rb2_ref[...]
    z = logits - jnp.max(logits, axis=-1, keepdims=True)
    ez = jnp.exp(z)
    probs = ez / jnp.sum(ez, axis=-1, keepdims=True)
    probs_ref[0] = probs

    ids = jax.lax.broadcasted_iota(jnp.int32, (1, N_EXP), 1)
    m1 = jnp.max(probs)
    i1 = jnp.min(jnp.where(probs == m1, ids, N_EXP))
    probs2 = jnp.where(ids == i1, -jnp.inf, probs)
    m2 = jnp.max(probs2)
    i2 = jnp.min(jnp.where(probs2 == m2, ids, N_EXP))
    denom = m1 + m2 + 1e-8
    pick = jax.lax.broadcasted_iota(jnp.int32, (1, TOP_K), 1)
    w_ref[0] = jnp.where(pick == 0, m1, m2) / denom
    idx_ref[0] = jnp.where(pick == 0, i1, i2).astype(jnp.int32)


def _run_router(h, rW1, rb1, rW2, rb2):
    probs, w, idx = pl.pallas_call(
        _router_kernel,
        grid=(BATCH, NSR),
        in_specs=[
            pl.BlockSpec((1, RBS, T_DIM), lambda b, s: (b, s, 0)),
            pl.BlockSpec((G_DIM, T_DIM), lambda b, s: (0, 0)),
            pl.BlockSpec((1, G_DIM), lambda b, s: (0, 0)),
            pl.BlockSpec((N_EXP, G_DIM), lambda b, s: (0, 0)),
            pl.BlockSpec((1, N_EXP), lambda b, s: (0, 0)),
        ],
        out_specs=[
            pl.BlockSpec((1, 1, N_EXP), lambda b, s: (b, 0, 0)),
            pl.BlockSpec((1, 1, TOP_K), lambda b, s: (b, 0, 0)),
            pl.BlockSpec((1, 1, TOP_K), lambda b, s: (b, 0, 0)),
        ],
        out_shape=[
            jax.ShapeDtypeStruct((BATCH, 1, N_EXP), f32),
            jax.ShapeDtypeStruct((BATCH, 1, TOP_K), f32),
            jax.ShapeDtypeStruct((BATCH, 1, TOP_K), jnp.int32),
        ],
        scratch_shapes=[pltpu.VMEM((1, T_DIM), f32)],
        compiler_params=pltpu.CompilerParams(
            dimension_semantics=("arbitrary", "arbitrary")),
    )(h, rW1, rb1.reshape(1, G_DIM), rW2, rb2.reshape(1, N_EXP))
    return probs.reshape(BATCH, N_EXP), w.reshape(BATCH, TOP_K), \
        idx.reshape(BATCH, TOP_K)


def _p_kernel(q_ref, bq_ref, wq_ref, wk_ref, p_ref):
    qq = jax.lax.dot_general(q_ref[...].astype(bf16),
                             wq_ref[...].astype(bf16), _DN,
                             preferred_element_type=f32) + bq_ref[...]
    rows = []
    for h_i in range(N_HEADS):
        sl = slice(h_i * HD, (h_i + 1) * HD)
        rows.append(jnp.dot(qq[:, sl].astype(bf16),
                            wk_ref[sl, :].astype(bf16),
                            preferred_element_type=f32))
    p_ref[...] = jnp.concatenate(rows, axis=0).astype(bf16)


def _run_p(q, bq, Win):
    return pl.pallas_call(
        _p_kernel,
        grid=(1,),
        in_specs=[
            pl.BlockSpec((N_TOK, S_DIM), lambda i: (0, 0)),
            pl.BlockSpec((1, S_DIM), lambda i: (0, 0)),
            pl.BlockSpec((S_DIM, S_DIM), lambda i: (0, 0)),
            pl.BlockSpec((S_DIM, S_DIM), lambda i: (1, 0)),
        ],
        out_specs=pl.BlockSpec((N_HEADS * N_TOK, S_DIM), lambda i: (0, 0)),
        out_shape=jax.ShapeDtypeStruct((N_HEADS * N_TOK, S_DIM), bf16),
        compiler_params=pltpu.CompilerParams(
            dimension_semantics=("arbitrary",)),
    )(q, bq, Win, Win)


def _mlp_kernel(se_ref, sb_ref,
                h_ref, wd_ref, bd_ref, wu_ref, bu_ref, lng_ref, lnb_ref,
                x2_ref):
    hb = h_ref[0].astype(bf16)
    x1 = jax.lax.dot_general(hb, wd_ref[0], _DN,
                             preferred_element_type=f32) + bd_ref[0]
    x1 = _gelu_exact(x1)
    x2 = jax.lax.dot_general(x1.astype(bf16), wu_ref[0], _DN,
                             preferred_element_type=f32) + bu_ref[0]
    x2 = _ln_f32(x2, lng_ref[0], lnb_ref[0])
    x2_ref[0] = x2.astype(bf16)


def _run_mlp(pair_expert, pair_batch, h,
             Wd_all, bd_all, Wu_all, bu_all, lng_all, lnb_all):
    grid_spec = pltpu.PrefetchScalarGridSpec(
        num_scalar_prefetch=2,
        grid=(N_PAIR, NS),
        in_specs=[
            pl.BlockSpec((1, BS, T_DIM), lambda p, s, se, sb: (sb[p], s, 0)),
            pl.BlockSpec((1, B_DIM, T_DIM), lambda p, s, se, sb: (se[p], 0, 0)),
            pl.BlockSpec((1, 1, B_DIM), lambda p, s, se, sb: (se[p], 0, 0)),
            pl.BlockSpec((1, S_DIM, B_DIM), lambda p, s, se, sb: (se[p], 0, 0)),
            pl.BlockSpec((1, 1, S_DIM), lambda p, s, se, sb: (se[p], 0, 0)),
            pl.BlockSpec((1, 1, S_DIM), lambda p, s, se, sb: (se[p], 0, 0)),
            pl.BlockSpec((1, 1, S_DIM), lambda p, s, se, sb: (se[p], 0, 0)),
        ],
        out_specs=pl.BlockSpec((1, BS, S_DIM),
                               lambda p, s, se, sb: (p, s, 0)),
        scratch_shapes=[],
    )
    return pl.pallas_call(
        _mlp_kernel,
        grid_spec=grid_spec,
        out_shape=jax.ShapeDtypeStruct((N_PAIR, SEQ, S_DIM), bf16),
        compiler_params=pltpu.CompilerParams(
            dimension_semantics=("arbitrary", "arbitrary")),
    )(pair_expert, pair_batch, h,
      Wd_all, bd_all, Wu_all, bu_all, lng_all, lnb_all)


def _attn_kernel(se_ref, sb_ref, w_ref,
                 x2_ref, p_ref, wv_ref, bv_ref, wo_ref, bo_ref, q_ref,
                 png_ref, pnb_ref, out_ref):
    p = pl.program_id(0)
    x2c = x2_ref[0]
    sc = jax.lax.dot_general(p_ref[0], x2c, _DN,
                             preferred_element_type=f32)
    ctx_heads = []
    for h_i in range(N_HEADS):
        att = jax.nn.softmax(sc[h_i * N_TOK:(h_i + 1) * N_TOK, :] / 16.0,
                             axis=-1)
        t = jax.lax.dot_general(att.astype(bf16), x2c,
                                (((1,), (0,)), ((), ())),
                                preferred_element_type=f32)
        wv_h = wv_ref[0][h_i * HD:(h_i + 1) * HD, :]
        ctx_heads.append(
            jax.lax.dot_general(t.astype(bf16), wv_h, _DN,
                                preferred_element_type=f32)
            + bv_ref[0][:, h_i * HD:(h_i + 1) * HD])
    ctx = jnp.concatenate(ctx_heads, axis=-1)
    val = jax.lax.dot_general(ctx.astype(bf16), wo_ref[0], _DN,
                              preferred_element_type=f32) + bo_ref[0]
    val = _ln_f32(val + q_ref[0], png_ref[0], pnb_ref[0]) * w_ref[p]

    first = (p != 3) & (p != 5)

    @pl.when(first)
    def _set():
        out_ref[0, 0] = val

    @pl.when(jnp.logical_not(first))
    def _acc():
        out_ref[0, 0] += val


def _run_attn(pair_expert, pair_batch, pair_w, x2_all, P_all,
              Wv_all, bv_all, Wo_all, bo_all, q_all, png_all, pnb_all):
    grid_spec = pltpu.PrefetchScalarGridSpec(
        num_scalar_prefetch=3,
        grid=(N_PAIR,),
        in_specs=[
            pl.BlockSpec((1, SEQ, S_DIM), lambda p, se, sb, w: (p, 0, 0)),
            pl.BlockSpec((1, N_HEADS * N_TOK, S_DIM),
                         lambda p, se, sb, w: (se[p], 0, 0)),
            pl.BlockSpec((1, S_DIM, S_DIM),
                         lambda p, se, sb, w: (se[p], 0, 0)),
            pl.BlockSpec((1, 1, S_DIM), lambda p, se, sb, w: (se[p], 0, 0)),
            pl.BlockSpec((1, S_DIM, S_DIM),
                         lambda p, se, sb, w: (se[p], 0, 0)),
            pl.BlockSpec((1, 1, S_DIM), lambda p, se, sb, w: (se[p], 0, 0)),
            pl.BlockSpec((1, N_TOK, S_DIM),
                         lambda p, se, sb, w: (se[p], 0, 0)),
            pl.BlockSpec((1, 1, S_DIM), lambda p, se, sb, w: (se[p], 0, 0)),
            pl.BlockSpec((1, 1, S_DIM), lambda p, se, sb, w: (se[p], 0, 0)),
        ],
        out_specs=pl.BlockSpec(
            (1, 1, N_TOK, S_DIM),
            lambda p, se, sb, w: (sb[p], jnp.where(se[p] == GEN_ID, 0, 1),
                                  0, 0)),
    )
    out4 = pl.pallas_call(
        _attn_kernel,
        grid_spec=grid_spec,
        out_shape=jax.ShapeDtypeStruct((BATCH, 2, N_TOK, S_DIM), f32),
        compiler_params=pltpu.CompilerParams(
            dimension_semantics=("arbitrary",)),
    )(pair_expert, pair_batch, pair_w, x2_all, P_all,
      Wv_all, bv_all, Wo_all, bo_all, q_all, png_all, pnb_all)
    return out4.reshape(BATCH, 2 * N_TOK, S_DIM)


def kernel(h_teacher, params):
    gen = params['gen']
    allp = list(params['spec']) + [gen]

    def stack(name, dtype=None, bias=False):
        a = jnp.stack([p[name] for p in allp])
        if bias:
            a = a[:, None, :]
        return a.astype(dtype) if dtype is not None else a

    Wd_all = stack('Wd', bf16)
    Wu_all = stack('Wu', bf16)
    bd_all = stack('bd', bias=True)
    bu_all = stack('bu', bias=True)
    lng_all = stack('ln_g', bias=True)
    lnb_all = stack('ln_b', bias=True)
    Wv_all = jnp.stack(
        [p['Win'][2 * S_DIM:3 * S_DIM] for p in allp]).astype(bf16)
    bv_all = jnp.stack([p['bin'][2 * S_DIM:3 * S_DIM] for p in allp])[:, None]
    q_all = jnp.stack([p['q'][0] for p in allp])
    Wo_all = stack('Wo', bf16)
    bo_all = stack('bo', bias=True)
    png_all = stack('pn_g', bias=True)
    pnb_all = stack('pn_b', bias=True)

    probs, w, idx = _run_router(h_teacher, params['rW1'], params['rb1'],
                                params['rW2'], params['rb2'])

    gen_i = jnp.full((1,), GEN_ID, jnp.int32)
    one_w = jnp.ones((1,), f32)
    pair_expert = jnp.concatenate([gen_i, gen_i, idx[0], idx[1]])
    pair_batch = jnp.array([0, 1, 0, 0, 1, 1], jnp.int32)
    pair_w = jnp.concatenate([one_w, one_w, w[0], w[1]])

    def p_for(p_):
        return _run_p(p_['q'][0], p_['bin'][None, :S_DIM], p_['Win'])

    P_list = []
    for e, p_ in enumerate(allp):
        if e == GEN_ID:
            P_list.append(p_for(p_))
        else:
            used = jnp.any(idx == e)
            P_list.append(jax.lax.cond(
                used,
                lambda p_=p_: p_for(p_),
                lambda: jnp.zeros((N_HEADS * N_TOK, S_DIM), bf16)))
    P_all = jnp.stack(P_list)

    x2_all = _run_mlp(pair_expert, pair_batch, h_teacher,
                      Wd_all, bd_all, Wu_all, bu_all, lng_all, lnb_all)
    c_agg = _run_attn(pair_expert, pair_batch, pair_w, x2_all, P_all,
                      Wv_all, bv_all, Wo_all, bo_all, q_all, png_all, pnb_all)
    return c_agg, probs


# speedup vs baseline: 1.3218x; 1.0011x over previous
import functools

import jax
import jax.numpy as jnp
from jax.experimental import pallas as pl
from jax.experimental.pallas import tpu as pltpu

T_DIM = 4096
S_DIM = 2048
B_DIM = 1024
N_TOK = 16
N_EXP = 8
TOP_K = 2
G_DIM = 512
N_HEADS = 8
HD = S_DIM // N_HEADS
SEQ = 2048
BATCH = 2

BS = 512
NS = SEQ // BS
N_PAIR = 2 * (1 + TOP_K)
GEN_ID = N_EXP

_DN = (((1,), (1,)), ((), ()))

f32 = jnp.float32
bf16 = jnp.bfloat16


def _gelu_exact(x):
    return 0.5 * x * (1.0 + jax.lax.erf(x * 0.7071067811865476))


def _ln_f32(x, g, b, eps=1e-5):
    mu = jnp.mean(x, axis=-1, keepdims=True)
    var = jnp.mean((x - mu) ** 2, axis=-1, keepdims=True)
    return (x - mu) / jnp.sqrt(var + eps) * g + b


RBS = 512
NSR = SEQ // RBS


def _router_kernel(h_ref, rw1_ref, rb1_ref, rw2_ref, rb2_ref,
                   probs_ref, w_ref, idx_ref, psum_scr):
    s = pl.program_id(1)

    @pl.when(s == 0)
    def _init():
        psum_scr[...] = jnp.zeros((1, T_DIM), f32)

    psum_scr[...] += jnp.sum(h_ref[0], axis=0, keepdims=True)

    @pl.when(s == NSR - 1)
    def _finish():
        _router_tail(psum_scr[...] / SEQ, rw1_ref, rb1_ref, rw2_ref, rb2_ref,
                     probs_ref, w_ref, idx_ref)


def _router_tail(pooled, rw1_ref, rb1_ref, rw2_ref, rb2_ref,
                 probs_ref, w_ref, idx_ref):
    hid = jax.lax.dot_general(pooled, rw1_ref[...], _DN,
                              preferred_element_type=f32) + rb1_ref[...]
    hid = _gelu_exact(hid)
    logits = jax.lax.dot_general(hid, rw2_ref[...], _DN,
                                 preferred_element_type=f32) + rb2_ref[...]
    z = logits - jnp.max(logits, axis=-1, keepdims=True)
    ez = jnp.exp(z)
    probs = ez / jnp.sum(ez, axis=-1, keepdims=True)
    probs_ref[0] = probs

    ids = jax.lax.broadcasted_iota(jnp.int32, (1, N_EXP), 1)
    m1 = jnp.max(probs)
    i1 = jnp.min(jnp.where(probs == m1, ids, N_EXP))
    probs2 = jnp.where(ids == i1, -jnp.inf, probs)
    m2 = jnp.max(probs2)
    i2 = jnp.min(jnp.where(probs2 == m2, ids, N_EXP))
    denom = m1 + m2 + 1e-8
    pick = jax.lax.broadcasted_iota(jnp.int32, (1, TOP_K), 1)
    w_ref[0] = jnp.where(pick == 0, m1, m2) / denom
    idx_ref[0] = jnp.where(pick == 0, i1, i2).astype(jnp.int32)


def _run_router(h, rW1, rb1, rW2, rb2):
    probs, w, idx = pl.pallas_call(
        _router_kernel,
        grid=(BATCH, NSR),
        in_specs=[
            pl.BlockSpec((1, RBS, T_DIM), lambda b, s: (b, s, 0)),
            pl.BlockSpec((G_DIM, T_DIM), lambda b, s: (0, 0)),
            pl.BlockSpec((1, G_DIM), lambda b, s: (0, 0)),
            pl.BlockSpec((N_EXP, G_DIM), lambda b, s: (0, 0)),
            pl.BlockSpec((1, N_EXP), lambda b, s: (0, 0)),
        ],
        out_specs=[
            pl.BlockSpec((1, 1, N_EXP), lambda b, s: (b, 0, 0)),
            pl.BlockSpec((1, 1, TOP_K), lambda b, s: (b, 0, 0)),
            pl.BlockSpec((1, 1, TOP_K), lambda b, s: (b, 0, 0)),
        ],
        out_shape=[
            jax.ShapeDtypeStruct((BATCH, 1, N_EXP), f32),
            jax.ShapeDtypeStruct((BATCH, 1, TOP_K), f32),
            jax.ShapeDtypeStruct((BATCH, 1, TOP_K), jnp.int32),
        ],
        scratch_shapes=[pltpu.VMEM((1, T_DIM), f32)],
        compiler_params=pltpu.CompilerParams(
            dimension_semantics=("arbitrary", "arbitrary")),
    )(h, rW1, rb1.reshape(1, G_DIM), rW2, rb2.reshape(1, N_EXP))
    return probs.reshape(BATCH, N_EXP), w.reshape(BATCH, TOP_K), \
        idx.reshape(BATCH, TOP_K)


def _p_kernel(q_ref, bq_ref, wq_ref, wk_ref, p_ref):
    qq = jax.lax.dot_general(q_ref[...].astype(bf16),
                             wq_ref[...].astype(bf16), _DN,
                             preferred_element_type=f32) + bq_ref[...]
    rows = []
    for h_i in range(N_HEADS):
        sl = slice(h_i * HD, (h_i + 1) * HD)
        rows.append(jnp.dot(qq[:, sl].astype(bf16),
                            wk_ref[sl, :].astype(bf16),
                            preferred_element_type=f32))
    p_ref[...] = jnp.concatenate(rows, axis=0).astype(bf16)


def _run_p(q, bq, Win):
    return pl.pallas_call(
        _p_kernel,
        grid=(1,),
        in_specs=[
            pl.BlockSpec((N_TOK, S_DIM), lambda i: (0, 0)),
            pl.BlockSpec((1, S_DIM), lambda i: (0, 0)),
            pl.BlockSpec((S_DIM, S_DIM), lambda i: (0, 0)),
            pl.BlockSpec((S_DIM, S_DIM), lambda i: (1, 0)),
        ],
        out_specs=pl.BlockSpec((N_HEADS * N_TOK, S_DIM), lambda i: (0, 0)),
        out_shape=jax.ShapeDtypeStruct((N_HEADS * N_TOK, S_DIM), bf16),
        compiler_params=pltpu.CompilerParams(
            dimension_semantics=("arbitrary",)),
    )(q, bq, Win, Win)


def _mlp_kernel(se_ref, sb_ref,
                h_ref, wd_ref, bd_ref, wu_ref, bu_ref, lng_ref, lnb_ref,
                x2_ref):
    hb = h_ref[0].astype(bf16)
    x1 = jax.lax.dot_general(hb, wd_ref[0], _DN,
                             preferred_element_type=f32) + bd_ref[0]
    x1 = _gelu_exact(x1)
    x2 = jax.lax.dot_general(x1.astype(bf16), wu_ref[0], _DN,
                             preferred_element_type=f32) + bu_ref[0]
    x2 = _ln_f32(x2, lng_ref[0], lnb_ref[0])
    x2_ref[0] = x2.astype(bf16)


def _run_mlp(pair_expert, pair_batch, h,
             Wd_all, bd_all, Wu_all, bu_all, lng_all, lnb_all):
    grid_spec = pltpu.PrefetchScalarGridSpec(
        num_scalar_prefetch=2,
        grid=(N_PAIR, NS),
        in_specs=[
            pl.BlockSpec((1, BS, T_DIM), lambda p, s, se, sb: (sb[p], s, 0)),
            pl.BlockSpec((1, B_DIM, T_DIM), lambda p, s, se, sb: (se[p], 0, 0)),
            pl.BlockSpec((1, 1, B_DIM), lambda p, s, se, sb: (se[p], 0, 0)),
            pl.BlockSpec((1, S_DIM, B_DIM), lambda p, s, se, sb: (se[p], 0, 0)),
            pl.BlockSpec((1, 1, S_DIM), lambda p, s, se, sb: (se[p], 0, 0)),
            pl.BlockSpec((1, 1, S_DIM), lambda p, s, se, sb: (se[p], 0, 0)),
            pl.BlockSpec((1, 1, S_DIM), lambda p, s, se, sb: (se[p], 0, 0)),
        ],
        out_specs=pl.BlockSpec((1, BS, S_DIM),
                               lambda p, s, se, sb: (p, s, 0)),
        scratch_shapes=[],
    )
    return pl.pallas_call(
        _mlp_kernel,
        grid_spec=grid_spec,
        out_shape=jax.ShapeDtypeStruct((N_PAIR, SEQ, S_DIM), bf16),
        compiler_params=pltpu.CompilerParams(
            dimension_semantics=("arbitrary", "arbitrary")),
    )(pair_expert, pair_batch, h,
      Wd_all, bd_all, Wu_all, bu_all, lng_all, lnb_all)


def _attn_kernel(se_ref, sb_ref, w_ref,
                 x2_ref, p_ref, wv_ref, bv_ref, wo_ref, bo_ref, q_ref,
                 png_ref, pnb_ref, out_ref):
    p = pl.program_id(0)
    x2c = x2_ref[0]
    sc = jax.lax.dot_general(p_ref[0], x2c, _DN,
                             preferred_element_type=f32)
    ctx_heads = []
    for h_i in range(N_HEADS):
        att = jax.nn.softmax(sc[h_i * N_TOK:(h_i + 1) * N_TOK, :] / 16.0,
                             axis=-1)
        t = jax.lax.dot_general(att.astype(bf16), x2c,
                                (((1,), (0,)), ((), ())),
                                preferred_element_type=f32)
        wv_h = wv_ref[0][h_i * HD:(h_i + 1) * HD, :]
        ctx_heads.append(
            jax.lax.dot_general(t.astype(bf16), wv_h, _DN,
                                preferred_element_type=f32)
            + bv_ref[0][:, h_i * HD:(h_i + 1) * HD])
    ctx = jnp.concatenate(ctx_heads, axis=-1)
    val = jax.lax.dot_general(ctx.astype(bf16), wo_ref[0], _DN,
                              preferred_element_type=f32) + bo_ref[0]
    val = _ln_f32(val + q_ref[0], png_ref[0], pnb_ref[0]) * w_ref[p]

    first = (p != 3) & (p != 5)

    @pl.when(first)
    def _set():
        out_ref[0, 0] = val

    @pl.when(jnp.logical_not(first))
    def _acc():
        out_ref[0, 0] += val


def _run_attn(pair_expert, pair_batch, pair_w, x2_all, P_all,
              Wv_all, bv_all, Wo_all, bo_all, q_all, png_all, pnb_all):
    grid_spec = pltpu.PrefetchScalarGridSpec(
        num_scalar_prefetch=3,
        grid=(N_PAIR,),
        in_specs=[
            pl.BlockSpec((1, SEQ, S_DIM), lambda p, se, sb, w: (p, 0, 0)),
            pl.BlockSpec((1, N_HEADS * N_TOK, S_DIM),
                         lambda p, se, sb, w: (se[p], 0, 0)),
            pl.BlockSpec((1, S_DIM, S_DIM),
                         lambda p, se, sb, w: (se[p], 0, 0)),
            pl.BlockSpec((1, 1, S_DIM), lambda p, se, sb, w: (se[p], 0, 0)),
            pl.BlockSpec((1, S_DIM, S_DIM),
                         lambda p, se, sb, w: (se[p], 0, 0)),
            pl.BlockSpec((1, 1, S_DIM), lambda p, se, sb, w: (se[p], 0, 0)),
            pl.BlockSpec((1, N_TOK, S_DIM),
                         lambda p, se, sb, w: (se[p], 0, 0)),
            pl.BlockSpec((1, 1, S_DIM), lambda p, se, sb, w: (se[p], 0, 0)),
            pl.BlockSpec((1, 1, S_DIM), lambda p, se, sb, w: (se[p], 0, 0)),
        ],
        out_specs=pl.BlockSpec(
            (1, 1, N_TOK, S_DIM),
            lambda p, se, sb, w: (sb[p], jnp.where(se[p] == GEN_ID, 0, 1),
                                  0, 0)),
    )
    out4 = pl.pallas_call(
        _attn_kernel,
        grid_spec=grid_spec,
        out_shape=jax.ShapeDtypeStruct((BATCH, 2, N_TOK, S_DIM), f32),
        compiler_params=pltpu.CompilerParams(
            dimension_semantics=("arbitrary",)),
    )(pair_expert, pair_batch, pair_w, x2_all, P_all,
      Wv_all, bv_all, Wo_all, bo_all, q_all, png_all, pnb_all)
    return out4.reshape(BATCH, 2 * N_TOK, S_DIM)


def kernel(h_teacher, params):
    gen = params['gen']
    allp = list(params['spec']) + [gen]

    def stack(name, dtype=None, bias=False):
        a = jnp.stack([p[name] for p in allp])
        if bias:
            a = a[:, None, :]
        return a.astype(dtype) if dtype is not None else a

    Wd_all = stack('Wd', bf16)
    Wu_all = stack('Wu', bf16)
    bd_all = stack('bd', bias=True)
    bu_all = stack('bu', bias=True)
    lng_all = stack('ln_g', bias=True)
    lnb_all = stack('ln_b', bias=True)
    Wv_all = jnp.stack(
        [p['Win'][2 * S_DIM:3 * S_DIM] for p in allp]).astype(bf16)
    bv_all = jnp.stack([p['bin'][2 * S_DIM:3 * S_DIM] for p in allp])[:, None]
    q_all = jnp.stack([p['q'][0] for p in allp])
    Wo_all = stack('Wo', bf16)
    bo_all = stack('bo', bias=True)
    png_all = stack('pn_g', bias=True)
    pnb_all = stack('pn_b', bias=True)

    probs, w, idx = _run_router(h_teacher, params['rW1'], params['rb1'],
                                params['rW2'], params['rb2'])

    gen_i = jnp.full((1,), GEN_ID, jnp.int32)
    one_w = jnp.ones((1,), f32)
    pair_expert = jnp.concatenate([gen_i, gen_i, idx[0], idx[1]])
    pair_batch = jnp.array([0, 1, 0, 0, 1, 1], jnp.int32)
    pair_w = jnp.concatenate([one_w, one_w, w[0], w[1]])

    def p_for(p_):
        return _run_p(p_['q'][0], p_['bin'][None, :S_DIM], p_['Win'])

    P_list = []
    for e, p_ in enumerate(allp):
        if e == GEN_ID:
            P_list.append(p_for(p_))
        else:
            used = jnp.any(idx == e)
            P_list.append(jax.lax.cond(
                used,
                lambda p_=p_: p_for(p_),
                lambda: jnp.zeros((N_HEADS * N_TOK, S_DIM), bf16)))
    P_all = jnp.stack(P_list)

    x2_all = _run_mlp(pair_expert, pair_batch, h_teacher,
                      Wd_all, bd_all, Wu_all, bu_all, lng_all, lnb_all)
    c_agg = _run_attn(pair_expert, pair_batch, pair_w, x2_all, P_all,
                      Wv_all, bv_all, Wo_all, bo_all, q_all, png_all, pnb_all)
    return c_agg, probs
